# baseline (speedup 1.0000x reference)
import jax
import jax.numpy as jnp
from jax.experimental import pallas as pl
from jax.experimental.pallas import tpu as pltpu

EMB_D = 64
K = 512
G = 32
B_TOTAL = 256
N_CHUNKS = B_TOTAL // G


def _compute_chunk(z_buf, es, e2, e_hi, e_lo, o_buf, slot):
    for g in range(G):
        z = z_buf[slot, g]
        d = e2 + jax.lax.dot_general(
            es, z, (((1,), (0,)), ((), ())),
            preferred_element_type=jnp.float32,
        )
        m = jnp.min(d, axis=0, keepdims=True)
        onehot = (d == m).astype(jnp.bfloat16)
        o_buf[slot, g] = jax.lax.dot_general(
            e_hi, onehot, (((0,), (0,)), ((), ())),
            preferred_element_type=jnp.float32,
        ) + jax.lax.dot_general(
            e_lo, onehot, (((0,), (0,)), ((), ())),
            preferred_element_type=jnp.float32,
        )


def _vq_kernel(z_hbm, e_ref, o_hbm, z_buf, o_buf, in_sem, out_sem):
    e = e_ref[...]
    es = e * -2.0
    e2 = jnp.sum(e * e, axis=1, keepdims=True)
    e_hi = e.astype(jnp.bfloat16)
    e_lo = (e - e_hi.astype(jnp.float32)).astype(jnp.bfloat16)

    def in_copy(i):
        return pltpu.make_async_copy(
            z_hbm.at[pl.ds(i * G, G)], z_buf.at[i % 2], in_sem.at[i % 2]
        )

    def out_copy(i):
        return pltpu.make_async_copy(
            o_buf.at[i % 2], o_hbm.at[pl.ds(i * G, G)], out_sem.at[i % 2]
        )

    in_copy(0).start()
    for i in range(N_CHUNKS):
        if i + 1 < N_CHUNKS:
            in_copy(i + 1).start()
        in_copy(i).wait()
        if i >= 2:
            out_copy(i - 2).wait()
        _compute_chunk(z_buf, es, e2, e_hi, e_lo, o_buf, i % 2)
        out_copy(i).start()
    out_copy(N_CHUNKS - 2).wait()
    out_copy(N_CHUNKS - 1).wait()


@jax.jit
def kernel(z_e, e):
    B, C, H, W = z_e.shape
    HW = H * W
    z = z_e.reshape(B, C, HW)
    out = pl.pallas_call(
        _vq_kernel,
        in_specs=[
            pl.BlockSpec(memory_space=pltpu.MemorySpace.HBM),
            pl.BlockSpec(memory_space=pltpu.MemorySpace.VMEM),
        ],
        out_specs=pl.BlockSpec(memory_space=pltpu.MemorySpace.HBM),
        out_shape=jax.ShapeDtypeStruct((B, C, HW), jnp.float32),
        scratch_shapes=[
            pltpu.VMEM((2, G, C, HW), jnp.float32),
            pltpu.VMEM((2, G, C, HW), jnp.float32),
            pltpu.SemaphoreType.DMA((2,)),
            pltpu.SemaphoreType.DMA((2,)),
        ],
    )(z, e)
    return out.reshape(B, C, H, W)
